# Initial kernel scaffold; baseline (speedup 1.0000x reference)
import jax
import jax.numpy as jnp
from jax.experimental import pallas as pl


def kernel(x, edge_index, W1, b1, W2, b2, W3, b3):
    raise NotImplementedError("write your pallas kernel here")


# speedup vs baseline: 25.1724x; 25.1724x over previous
import functools

import jax
import jax.numpy as jnp
from jax import lax
from jax.experimental import pallas as pl
from jax.experimental.pallas import tpu as pltpu
from jax.experimental.pallas import tpu_sc as plsc

N = 10000
NC, NS = 2, 16
NW = NC * NS
NP = 10112
RPS = NP // NS
CHUNK = 80

_MESH = dict(core_axis_name="c", subcore_axis_name="s",
             num_cores=NC, num_subcores=NS)
_SC_PARAMS = pltpu.CompilerParams(needs_layout_passes=False)


DR = 640
DRS = DR // NS


def _deg_body(dst_hbm, zeros_hbm, iota_hbm, out_hbm,
              dstf_v, local_v, iota_v, deg_sh, ew):
    c = lax.axis_index("c")
    s = lax.axis_index("s")
    wid = c * NS + s
    pltpu.sync_copy(zeros_hbm.at[pl.ds(0, DRS)],
                    deg_sh.at[pl.ds(s * DRS, DRS)])
    pltpu.sync_copy(zeros_hbm, local_v)
    pltpu.sync_copy(iota_hbm, iota_v)
    pltpu.sync_copy(dst_hbm.at[wid], dstf_v)

    ones = jnp.ones((16,), jnp.float32)

    def hist(i, _):
        idx = dstf_v[pl.ds(i * 16, 16)]
        hi = lax.shift_right_logical(idx, 4)
        lo = jnp.bitwise_and(idx, 15)
        plsc.addupdate_scatter(local_v, [hi, lo], ones)
        return 0

    lax.fori_loop(0, ew // 16, hist, 0)
    plsc.subcore_barrier()
    for j in range(DR // 128):
        pltpu.sync_copy(local_v.at[pl.ds(j * 128, 128)],
                        deg_sh.at[iota_v.at[j]], add=True)
    plsc.subcore_barrier()
    pltpu.sync_copy(deg_sh.at[pl.ds(s * DRS, DRS)],
                    out_hbm.at[c, pl.ds(s * DRS, DRS)])


def _make_deg_kernel(ew):
    return functools.partial(
        pl.kernel,
        out_type=jax.ShapeDtypeStruct((NC, DR, 16), jnp.float32),
        mesh=plsc.VectorSubcoreMesh(**_MESH),
        compiler_params=_SC_PARAMS,
        scratch_types=[
            pltpu.VMEM((ew,), jnp.int32),
            pltpu.VMEM((DR, 16), jnp.float32),
            pltpu.VMEM((DR // 128, 128), jnp.int32),
            pltpu.VMEM_SHARED((DR, 16), jnp.float32),
        ],
    )(functools.partial(_deg_body, ew=ew))


def _edge_body(src_hbm, dst_hbm, g_hbm, zeros_hbm, out_hbm,
               srcv, dstv, rows0, rows1, acc_sh, sem0, sem1, *, nch, d):
    c = lax.axis_index("c")
    s = lax.axis_index("s")
    wid = c * NS + s
    pltpu.sync_copy(src_hbm.at[wid], srcv)
    pltpu.sync_copy(dst_hbm.at[wid], dstv)
    pltpu.sync_copy(zeros_hbm, acc_sh.at[pl.ds(s * RPS, RPS)])
    plsc.subcore_barrier()

    rows = (rows0, rows1)
    sems = (sem0, sem1)

    def src_at(i):
        return srcv.at[pl.ds(i * CHUNK, CHUNK)]

    pltpu.async_copy(g_hbm.at[src_at(0)], rows0, sem0)
    pltpu.async_copy(g_hbm.at[src_at(1)], rows1, sem1)

    def outer(i0, _):
        for b in range(2):
            i = i0 * 2 + b

            @pl.when(i < nch)
            def _():
                pltpu.make_async_copy(g_hbm.at[src_at(i)], rows[b],
                                      sems[b]).wait()
                pltpu.sync_copy(rows[b], acc_sh.at[dstv.at[i]], add=True)

                @pl.when(i + 2 < nch)
                def _():
                    pltpu.async_copy(g_hbm.at[src_at(i + 2)], rows[b],
                                     sems[b])
        return 0

    lax.fori_loop(0, (nch + 1) // 2, outer, 0)
    plsc.subcore_barrier()
    pltpu.sync_copy(acc_sh.at[pl.ds(s * RPS, RPS)],
                    out_hbm.at[c, pl.ds(s * RPS, RPS)])


def _make_edge_kernel(nch, d):
    return functools.partial(
        pl.kernel,
        out_type=jax.ShapeDtypeStruct((NC, NP, d), jnp.float32),
        mesh=plsc.VectorSubcoreMesh(**_MESH),
        compiler_params=_SC_PARAMS,
        scratch_types=[
            pltpu.VMEM((nch * CHUNK,), jnp.int32),
            pltpu.VMEM((nch, CHUNK), jnp.int32),
            pltpu.VMEM((CHUNK, d), jnp.float32),
            pltpu.VMEM((CHUNK, d), jnp.float32),
            pltpu.VMEM_SHARED((NP, d), jnp.float32),
            pltpu.SemaphoreType.DMA,
            pltpu.SemaphoreType.DMA,
        ],
    )(functools.partial(_edge_body, nch=nch, d=d))


BLK = 1000


def _dis(deg_ref):
    deg = deg_ref[0] + deg_ref[1] + 1.0
    return lax.rsqrt(deg)


def _mm_first_body(deg_ref, x_ref, w_ref, o_ref):
    o_ref[...] = _dis(deg_ref) * jnp.dot(
        x_ref[...], w_ref[...], preferred_element_type=jnp.float32)


def _mm_mid_body(deg_ref, acc_ref, g_ref, b_ref, w_ref, o_ref):
    dis = _dis(deg_ref)
    pre = dis * (acc_ref[0] + acc_ref[1] + g_ref[...]) + b_ref[...]
    act = jnp.maximum(pre, 0.0)
    o_ref[...] = dis * jnp.dot(act, w_ref[...],
                               preferred_element_type=jnp.float32)


def _mm_final_body(deg_ref, acc_ref, g_ref, b_ref, o_ref):
    dis = _dis(deg_ref)
    res = dis * (acc_ref[0] + acc_ref[1] + g_ref[...])
    o_ref[...] = res[:, :b_ref.shape[1]] + b_ref[...]


def _deg_spec():
    return pl.BlockSpec((NC, BLK, 1), lambda i: (0, i, 0))


def _acc_spec(d):
    return pl.BlockSpec((NC, BLK, d), lambda i: (0, i, 0))


def _row_spec(d):
    return pl.BlockSpec((BLK, d), lambda i: (i, 0))


def _full_spec(r, c):
    return pl.BlockSpec((r, c), lambda i: (0, 0))


def _mm_first(degp, x, w):
    return pl.pallas_call(
        _mm_first_body,
        grid=(N // BLK,),
        in_specs=[_deg_spec(), _row_spec(x.shape[1]), _full_spec(*w.shape)],
        out_specs=_row_spec(w.shape[1]),
        out_shape=jax.ShapeDtypeStruct((N, w.shape[1]), jnp.float32),
    )(degp, x, w)


def _mm_mid(degp, acc, g, b, w):
    d_in, d_out = w.shape
    return pl.pallas_call(
        _mm_mid_body,
        grid=(N // BLK,),
        in_specs=[_deg_spec(), _acc_spec(d_in), _row_spec(d_in),
                  _full_spec(1, d_in), _full_spec(d_in, d_out)],
        out_specs=_row_spec(d_out),
        out_shape=jax.ShapeDtypeStruct((N, d_out), jnp.float32),
    )(degp, acc, g, b, w)


def _mm_final(degp, acc, g, b):
    d = g.shape[1]
    d_out = b.shape[1]
    return pl.pallas_call(
        _mm_final_body,
        grid=(N // BLK,),
        in_specs=[_deg_spec(), _acc_spec(d), _row_spec(d),
                  _full_spec(1, d_out)],
        out_specs=_row_spec(d_out),
        out_shape=jax.ShapeDtypeStruct((N, d_out), jnp.float32),
    )(degp, acc, g, b)


def kernel(x, edge_index, W1, b1, W2, b2, W3, b3):
    assert x.shape[0] == N
    e = edge_index.shape[1]
    ew = e // NW
    assert e == ew * NW and ew % CHUNK == 0 and ew % 16 == 0
    nch = ew // CHUNK

    ei = edge_index.astype(jnp.int32)
    src = ei[0].reshape(NW, ew)
    dst = ei[1].reshape(NW, nch, CHUNK)
    dstf = ei[1].reshape(NW, ew)

    z16 = jnp.zeros((DR, 16), jnp.float32)
    z128 = jnp.zeros((RPS, 128), jnp.float32)
    iota = jnp.arange(DR, dtype=jnp.int32).reshape(DR // 128, 128)

    deg_k = _make_deg_kernel(ew)
    edge128 = _make_edge_kernel(nch, 128)

    degp = deg_k(dstf, z16, iota).reshape(NC, DR * 16, 1)

    g1 = _mm_first(degp, x, W1)
    acc1 = edge128(src, dst, g1, z128)
    g2 = _mm_mid(degp, acc1, g1, b1.reshape(1, 128), W2)
    acc2 = edge128(src, dst, g2, z128)
    w3p = jnp.zeros((128, 128), jnp.float32).at[:, :40].set(W3)
    g3 = _mm_mid(degp, acc2, g2, b2.reshape(1, 128), w3p)
    acc3 = edge128(src, dst, g3, z128)
    return _mm_final(degp, acc3, g3, b3.reshape(1, 40))
